# baseline (speedup 1.0000x reference)
import functools
import math

import jax
import jax.numpy as jnp
from jax import lax
from jax.experimental import pallas as pl
from jax.experimental.pallas import tpu as pltpu
from jax.experimental.pallas import tpu_sc as plsc

_ALPHA = 0.2
_THETA = 1.0
_NUM_LAYERS = 8

_KD = 40
_KS = 80
_WD = 16
_CH = 128


_GRP = 25


@functools.cache
def _make_spmm_kernel(n_nodes: int, n_edges: int):
    nrows = n_edges // _KS
    rows_t = nrows // 16
    ngrp = rows_t // _GRP
    npt = n_nodes // 16
    mesh = plsc.VectorSubcoreMesh(core_axis_name="c", subcore_axis_name="s")

    @functools.partial(
        pl.kernel,
        out_type=[jax.ShapeDtypeStruct((16, npt, _CH), jnp.float32),
                  jax.ShapeDtypeStruct((16, npt, _CH), jnp.float32)],
        mesh=mesh,
        scratch_types=[
            pltpu.VMEM((2, _GRP, _KS), jnp.int32),
            pltpu.VMEM((2, _GRP, _KS), jnp.int32),
            pltpu.VMEM((2, _KS, _CH), jnp.float32),
            pltpu.VMEM_SHARED((n_nodes, _CH), jnp.float32),
            pltpu.SemaphoreType.DMA((2,)),
            pltpu.SemaphoreType.DMA((2,)),
            pltpu.SemaphoreType.DMA((2,)),
        ],
    )
    def spmm_kernel(src_hbm, dst_hbm, lo_hbm, hi_hbm, z_hbm,
                    alo_hbm, ahi_hbm, srcv, dstv, rows_v, acc_sh,
                    isems, rsems, ssems):
        c = lax.axis_index("c")
        s = lax.axis_index("s")
        pltpu.sync_copy(z_hbm.at[s], acc_sh.at[pl.ds(s * npt, npt)])
        plsc.subcore_barrier()

        def run(tab_hbm):
            pltpu.async_copy(src_hbm.at[s].at[0], srcv.at[0], isems.at[0])
            pltpu.async_copy(dst_hbm.at[s].at[0], dstv.at[0], isems.at[0])

            def group(g, carry):
                gb = lax.rem(g, 2)
                gnb = lax.rem(g + 1, 2)

                @pl.when(g + 1 < ngrp)
                def _():
                    pltpu.async_copy(src_hbm.at[s].at[g + 1], srcv.at[gnb],
                                     isems.at[gnb])
                    pltpu.async_copy(dst_hbm.at[s].at[g + 1], dstv.at[gnb],
                                     isems.at[gnb])

                pltpu.make_async_copy(src_hbm.at[s].at[g], srcv.at[gb],
                                      isems.at[gb]).wait()
                pltpu.make_async_copy(dst_hbm.at[s].at[g], dstv.at[gb],
                                      isems.at[gb]).wait()
                pltpu.async_copy(tab_hbm.at[srcv.at[gb].at[0]], rows_v.at[0],
                                 rsems.at[0])

                def chunk(i, carry2):
                    b = lax.rem(i, 2)
                    nb = lax.rem(i + 1, 2)
                    pltpu.make_async_copy(tab_hbm.at[srcv.at[gb].at[i]],
                                          rows_v.at[b], rsems.at[b]).wait()
                    pltpu.async_copy(rows_v.at[b],
                                     acc_sh.at[dstv.at[gb].at[i]],
                                     ssems.at[b], add=True)

                    @pl.when(i + 1 < _GRP)
                    def _():
                        @pl.when(i >= 1)
                        def _():
                            pltpu.make_async_copy(
                                rows_v.at[nb],
                                acc_sh.at[dstv.at[gb].at[i - 1]],
                                ssems.at[nb]).wait()

                        pltpu.async_copy(tab_hbm.at[srcv.at[gb].at[i + 1]],
                                         rows_v.at[nb], rsems.at[nb])

                    return carry2

                lax.fori_loop(0, _GRP, chunk, 0)
                pltpu.make_async_copy(rows_v.at[(_GRP - 2) % 2],
                                      acc_sh.at[dstv.at[gb].at[_GRP - 2]],
                                      ssems.at[(_GRP - 2) % 2]).wait()
                pltpu.make_async_copy(rows_v.at[(_GRP - 1) % 2],
                                      acc_sh.at[dstv.at[gb].at[_GRP - 1]],
                                      ssems.at[(_GRP - 1) % 2]).wait()
                return carry

            lax.fori_loop(0, ngrp, group, 0)

        @pl.when(c == 0)
        def _():
            run(lo_hbm)

        @pl.when(c == 1)
        def _():
            run(hi_hbm)

        plsc.subcore_barrier()

        @pl.when(c == 0)
        def _():
            pltpu.sync_copy(acc_sh.at[pl.ds(s * npt, npt)], alo_hbm.at[s])

        @pl.when(c == 1)
        def _():
            pltpu.sync_copy(acc_sh.at[pl.ds(s * npt, npt)], ahi_hbm.at[s])

    return spmm_kernel


@functools.cache
def _make_prologue(n: int, d: int, h: int, blk: int):
    def body(x_ref, wt_ref, b_ref, deg_ref,
             x0_ref, lo_ref, hi_ref, dinv_ref):
        hm = jnp.dot(x_ref[...], wt_ref[...],
                     preferred_element_type=jnp.float32) + b_ref[...]
        hm = jnp.maximum(hm, 0.0)
        deg = deg_ref[:, 0:1] + 1.0
        dinv = lax.rsqrt(deg)
        x0_ref[...] = hm
        op = hm * dinv
        lo_ref[...] = op[:, :_CH]
        hi_ref[...] = op[:, _CH:]
        dinv_ref[...] = dinv

    grid = (n // blk,)
    return pl.pallas_call(
        body,
        grid=grid,
        in_specs=[
            pl.BlockSpec((blk, d), lambda i: (i, 0)),
            pl.BlockSpec((d, h), lambda i: (0, 0)),
            pl.BlockSpec((1, h), lambda i: (0, 0)),
            pl.BlockSpec((blk, _CH), lambda i: (i, 0)),
        ],
        out_specs=[
            pl.BlockSpec((blk, h), lambda i: (i, 0)),
            pl.BlockSpec((blk, _CH), lambda i: (i, 0)),
            pl.BlockSpec((blk, _CH), lambda i: (i, 0)),
            pl.BlockSpec((blk, 1), lambda i: (i, 0)),
        ],
        out_shape=[
            jax.ShapeDtypeStruct((n, h), jnp.float32),
            jax.ShapeDtypeStruct((n, _CH), jnp.float32),
            jax.ShapeDtypeStruct((n, _CH), jnp.float32),
            jax.ShapeDtypeStruct((n, 1), jnp.float32),
        ],
    )


@functools.cache
def _make_layer(n: int, h: int, beta: float, last: bool, blk: int):
    def body(alo_ref, ahi_ref, lo_ref, hi_ref, x0_ref, dinv_ref, w_ref, *outs):
        agg = jnp.concatenate(
            [alo_ref[...] + lo_ref[...], ahi_ref[...] + hi_ref[...]], axis=1)
        dinv = dinv_ref[...]
        hc = (1.0 - _ALPHA) * (agg * dinv) + _ALPHA * x0_ref[...]
        out = (1.0 - beta) * hc + beta * jnp.dot(
            hc, w_ref[...], preferred_element_type=jnp.float32)
        if last:
            outs[0][...] = out
        else:
            op = jnp.maximum(out, 0.0) * dinv
            outs[0][...] = op[:, :_CH]
            outs[1][...] = op[:, _CH:]

    grid = (n // blk,)
    in_specs = [
        pl.BlockSpec((blk, _CH), lambda i: (i, 0)),
        pl.BlockSpec((blk, _CH), lambda i: (i, 0)),
        pl.BlockSpec((blk, _CH), lambda i: (i, 0)),
        pl.BlockSpec((blk, _CH), lambda i: (i, 0)),
        pl.BlockSpec((blk, h), lambda i: (i, 0)),
        pl.BlockSpec((blk, 1), lambda i: (i, 0)),
        pl.BlockSpec((h, h), lambda i: (0, 0)),
    ]
    if last:
        out_specs = [pl.BlockSpec((blk, h), lambda i: (i, 0))]
        out_shape = [jax.ShapeDtypeStruct((n, h), jnp.float32)]
    else:
        out_specs = [pl.BlockSpec((blk, _CH), lambda i: (i, 0)),
                     pl.BlockSpec((blk, _CH), lambda i: (i, 0))]
        out_shape = [jax.ShapeDtypeStruct((n, _CH), jnp.float32),
                     jax.ShapeDtypeStruct((n, _CH), jnp.float32)]
    return pl.pallas_call(
        body, grid=grid, in_specs=in_specs, out_specs=out_specs,
        out_shape=out_shape)


def kernel(x, edge_index, lin_w, lin_b, conv_ws):
    n, d = x.shape
    e = edge_index.shape[1]
    h = lin_w.shape[0]
    blk = 1000

    npt = n // 16
    rows_t = e // (16 * _KS)
    src3d = edge_index[0].reshape(16, rows_t // _GRP, _GRP, _KS)
    dst3d = edge_index[1].reshape(16, rows_t // _GRP, _GRP, _KS)
    zeros_c = jnp.zeros((16, npt, _CH), jnp.float32)
    ones_c = jnp.ones((n, _CH), jnp.float32)

    spmm = _make_spmm_kernel(n, e)
    dego, _ = spmm(src3d, dst3d, ones_c, ones_c, zeros_c)
    x0, lo, hi, dinv = _make_prologue(n, d, h, blk)(
        x, lin_w.T, lin_b.reshape(1, h), dego.reshape(n, _CH))
    out = None
    for layer in range(_NUM_LAYERS):
        beta = math.log(_THETA / (layer + 1) + 1.0)
        alo, ahi = spmm(src3d, dst3d, lo, hi, zeros_c)
        last = layer == _NUM_LAYERS - 1
        layer_fn = _make_layer(n, h, beta, last, blk)
        args = (alo.reshape(n, _CH), ahi.reshape(n, _CH), lo, hi, x0, dinv,
                conv_ws[layer])
        if last:
            (out,) = layer_fn(*args)
        else:
            lo, hi = layer_fn(*args)
    return out


# speedup vs baseline: 1.2976x; 1.2976x over previous
import functools
import math

import jax
import jax.numpy as jnp
from jax import lax
from jax.experimental import pallas as pl
from jax.experimental.pallas import tpu as pltpu
from jax.experimental.pallas import tpu_sc as plsc

_ALPHA = 0.2
_THETA = 1.0
_NUM_LAYERS = 8

_KD = 40
_KS = 100
_WD = 16
_CH = 128


_GRP = 25


@functools.cache
def _make_spmm_kernel(n_nodes: int, n_edges: int):
    nrows = n_edges // _KS
    rows_t = nrows // 16
    ngrp = rows_t // _GRP
    npt = n_nodes // 16
    mesh = plsc.VectorSubcoreMesh(core_axis_name="c", subcore_axis_name="s")

    @functools.partial(
        pl.kernel,
        out_type=[jax.ShapeDtypeStruct((16, npt, _CH), jnp.float32),
                  jax.ShapeDtypeStruct((16, npt, _CH), jnp.float32)],
        mesh=mesh,
        scratch_types=[
            pltpu.VMEM((2, _GRP, _KS), jnp.int32),
            pltpu.VMEM((2, _GRP, _KS), jnp.int32),
            pltpu.VMEM((2, _KS, _CH), jnp.float32),
            pltpu.VMEM_SHARED((n_nodes, _CH), jnp.float32),
            pltpu.SemaphoreType.DMA((2,)),
            pltpu.SemaphoreType.DMA((2,)),
            pltpu.SemaphoreType.DMA((2,)),
        ],
    )
    def spmm_kernel(src_hbm, dst_hbm, lo_hbm, hi_hbm, z_hbm,
                    alo_hbm, ahi_hbm, srcv, dstv, rows_v, acc_sh,
                    isems, rsems, ssems):
        c = lax.axis_index("c")
        s = lax.axis_index("s")
        pltpu.sync_copy(z_hbm.at[s], acc_sh.at[pl.ds(s * npt, npt)])
        plsc.subcore_barrier()

        def run(tab_hbm):
            pltpu.async_copy(src_hbm.at[s].at[0], srcv.at[0], isems.at[0])
            pltpu.async_copy(dst_hbm.at[s].at[0], dstv.at[0], isems.at[0])

            def group(g, carry):
                gb = lax.rem(g, 2)
                gnb = lax.rem(g + 1, 2)

                @pl.when(g + 1 < ngrp)
                def _():
                    pltpu.async_copy(src_hbm.at[s].at[g + 1], srcv.at[gnb],
                                     isems.at[gnb])
                    pltpu.async_copy(dst_hbm.at[s].at[g + 1], dstv.at[gnb],
                                     isems.at[gnb])

                pltpu.make_async_copy(src_hbm.at[s].at[g], srcv.at[gb],
                                      isems.at[gb]).wait()
                pltpu.make_async_copy(dst_hbm.at[s].at[g], dstv.at[gb],
                                      isems.at[gb]).wait()
                pltpu.async_copy(tab_hbm.at[srcv.at[gb].at[0]], rows_v.at[0],
                                 rsems.at[0])

                def chunk(i, carry2):
                    b = lax.rem(i, 2)
                    nb = lax.rem(i + 1, 2)

                    @pl.when(i + 1 < _GRP)
                    def _():
                        pltpu.async_copy(tab_hbm.at[srcv.at[gb].at[i + 1]],
                                         rows_v.at[nb], rsems.at[nb])

                    pltpu.make_async_copy(tab_hbm.at[srcv.at[gb].at[i]],
                                          rows_v.at[b], rsems.at[b]).wait()
                    pltpu.sync_copy(rows_v.at[b],
                                    acc_sh.at[dstv.at[gb].at[i]], add=True)
                    return carry2

                lax.fori_loop(0, _GRP, chunk, 0)
                return carry

            lax.fori_loop(0, ngrp, group, 0)

        @pl.when(c == 0)
        def _():
            run(lo_hbm)

        @pl.when(c == 1)
        def _():
            run(hi_hbm)

        plsc.subcore_barrier()

        @pl.when(c == 0)
        def _():
            pltpu.sync_copy(acc_sh.at[pl.ds(s * npt, npt)], alo_hbm.at[s])

        @pl.when(c == 1)
        def _():
            pltpu.sync_copy(acc_sh.at[pl.ds(s * npt, npt)], ahi_hbm.at[s])

    return spmm_kernel


@functools.cache
def _make_prologue(n: int, d: int, h: int, blk: int):
    def body(x_ref, wt_ref, b_ref, deg_ref,
             x0_ref, lo_ref, hi_ref, dinv_ref):
        hm = jnp.dot(x_ref[...], wt_ref[...],
                     preferred_element_type=jnp.float32) + b_ref[...]
        hm = jnp.maximum(hm, 0.0)
        deg = deg_ref[:, 0:1] + 1.0
        dinv = lax.rsqrt(deg)
        x0_ref[...] = hm
        op = hm * dinv
        lo_ref[...] = op[:, :_CH]
        hi_ref[...] = op[:, _CH:]
        dinv_ref[...] = dinv

    grid = (n // blk,)
    return pl.pallas_call(
        body,
        grid=grid,
        in_specs=[
            pl.BlockSpec((blk, d), lambda i: (i, 0)),
            pl.BlockSpec((d, h), lambda i: (0, 0)),
            pl.BlockSpec((1, h), lambda i: (0, 0)),
            pl.BlockSpec((blk, _CH), lambda i: (i, 0)),
        ],
        out_specs=[
            pl.BlockSpec((blk, h), lambda i: (i, 0)),
            pl.BlockSpec((blk, _CH), lambda i: (i, 0)),
            pl.BlockSpec((blk, _CH), lambda i: (i, 0)),
            pl.BlockSpec((blk, 1), lambda i: (i, 0)),
        ],
        out_shape=[
            jax.ShapeDtypeStruct((n, h), jnp.float32),
            jax.ShapeDtypeStruct((n, _CH), jnp.float32),
            jax.ShapeDtypeStruct((n, _CH), jnp.float32),
            jax.ShapeDtypeStruct((n, 1), jnp.float32),
        ],
    )


@functools.cache
def _make_layer(n: int, h: int, beta: float, last: bool, blk: int):
    def body(alo_ref, ahi_ref, lo_ref, hi_ref, x0_ref, dinv_ref, w_ref, *outs):
        agg = jnp.concatenate(
            [alo_ref[...] + lo_ref[...], ahi_ref[...] + hi_ref[...]], axis=1)
        dinv = dinv_ref[...]
        hc = (1.0 - _ALPHA) * (agg * dinv) + _ALPHA * x0_ref[...]
        out = (1.0 - beta) * hc + beta * jnp.dot(
            hc, w_ref[...], preferred_element_type=jnp.float32)
        if last:
            outs[0][...] = out
        else:
            op = jnp.maximum(out, 0.0) * dinv
            outs[0][...] = op[:, :_CH]
            outs[1][...] = op[:, _CH:]

    grid = (n // blk,)
    in_specs = [
        pl.BlockSpec((blk, _CH), lambda i: (i, 0)),
        pl.BlockSpec((blk, _CH), lambda i: (i, 0)),
        pl.BlockSpec((blk, _CH), lambda i: (i, 0)),
        pl.BlockSpec((blk, _CH), lambda i: (i, 0)),
        pl.BlockSpec((blk, h), lambda i: (i, 0)),
        pl.BlockSpec((blk, 1), lambda i: (i, 0)),
        pl.BlockSpec((h, h), lambda i: (0, 0)),
    ]
    if last:
        out_specs = [pl.BlockSpec((blk, h), lambda i: (i, 0))]
        out_shape = [jax.ShapeDtypeStruct((n, h), jnp.float32)]
    else:
        out_specs = [pl.BlockSpec((blk, _CH), lambda i: (i, 0)),
                     pl.BlockSpec((blk, _CH), lambda i: (i, 0))]
        out_shape = [jax.ShapeDtypeStruct((n, _CH), jnp.float32),
                     jax.ShapeDtypeStruct((n, _CH), jnp.float32)]
    return pl.pallas_call(
        body, grid=grid, in_specs=in_specs, out_specs=out_specs,
        out_shape=out_shape)


def kernel(x, edge_index, lin_w, lin_b, conv_ws):
    n, d = x.shape
    e = edge_index.shape[1]
    h = lin_w.shape[0]
    blk = 1000

    npt = n // 16
    rows_t = e // (16 * _KS)
    src3d = edge_index[0].reshape(16, rows_t // _GRP, _GRP, _KS)
    dst3d = edge_index[1].reshape(16, rows_t // _GRP, _GRP, _KS)
    zeros_c = jnp.zeros((16, npt, _CH), jnp.float32)
    ones_c = jnp.ones((n, _CH), jnp.float32)

    spmm = _make_spmm_kernel(n, e)
    dego, _ = spmm(src3d, dst3d, ones_c, ones_c, zeros_c)
    x0, lo, hi, dinv = _make_prologue(n, d, h, blk)(
        x, lin_w.T, lin_b.reshape(1, h), dego.reshape(n, _CH))
    out = None
    for layer in range(_NUM_LAYERS):
        beta = math.log(_THETA / (layer + 1) + 1.0)
        alo, ahi = spmm(src3d, dst3d, lo, hi, zeros_c)
        last = layer == _NUM_LAYERS - 1
        layer_fn = _make_layer(n, h, beta, last, blk)
        args = (alo.reshape(n, _CH), ahi.reshape(n, _CH), lo, hi, x0, dinv,
                conv_ws[layer])
        if last:
            (out,) = layer_fn(*args)
        else:
            lo, hi = layer_fn(*args)
    return out


# speedup vs baseline: 1.3387x; 1.0316x over previous
import functools
import math

import jax
import jax.numpy as jnp
from jax import lax
from jax.experimental import pallas as pl
from jax.experimental.pallas import tpu as pltpu
from jax.experimental.pallas import tpu_sc as plsc

_ALPHA = 0.2
_THETA = 1.0
_NUM_LAYERS = 8

_KD = 40
_KS = 125
_WD = 16
_CH = 128


_GRP = 20


@functools.cache
def _make_spmm_kernel(n_nodes: int, n_edges: int):
    nrows = n_edges // _KS
    rows_t = nrows // 16
    ngrp = rows_t // _GRP
    npt = n_nodes // 16
    mesh = plsc.VectorSubcoreMesh(core_axis_name="c", subcore_axis_name="s")

    @functools.partial(
        pl.kernel,
        out_type=[jax.ShapeDtypeStruct((16, npt, _CH), jnp.float32),
                  jax.ShapeDtypeStruct((16, npt, _CH), jnp.float32)],
        mesh=mesh,
        scratch_types=[
            pltpu.VMEM((2, _GRP, _KS), jnp.int32),
            pltpu.VMEM((2, _GRP, _KS), jnp.int32),
            pltpu.VMEM((2, _KS, _CH), jnp.float32),
            pltpu.VMEM_SHARED((n_nodes, _CH), jnp.float32),
            pltpu.SemaphoreType.DMA((2,)),
            pltpu.SemaphoreType.DMA((2,)),
            pltpu.SemaphoreType.DMA((2,)),
        ],
    )
    def spmm_kernel(src_hbm, dst_hbm, lo_hbm, hi_hbm, z_hbm,
                    alo_hbm, ahi_hbm, srcv, dstv, rows_v, acc_sh,
                    isems, rsems, ssems):
        c = lax.axis_index("c")
        s = lax.axis_index("s")
        pltpu.sync_copy(z_hbm.at[s], acc_sh.at[pl.ds(s * npt, npt)])
        plsc.subcore_barrier()

        def run(tab_hbm):
            pltpu.async_copy(src_hbm.at[s].at[0], srcv.at[0], isems.at[0])
            pltpu.async_copy(dst_hbm.at[s].at[0], dstv.at[0], isems.at[0])

            def group(g, carry):
                gb = lax.rem(g, 2)
                gnb = lax.rem(g + 1, 2)

                @pl.when(g + 1 < ngrp)
                def _():
                    pltpu.async_copy(src_hbm.at[s].at[g + 1], srcv.at[gnb],
                                     isems.at[gnb])
                    pltpu.async_copy(dst_hbm.at[s].at[g + 1], dstv.at[gnb],
                                     isems.at[gnb])

                pltpu.make_async_copy(src_hbm.at[s].at[g], srcv.at[gb],
                                      isems.at[gb]).wait()
                pltpu.make_async_copy(dst_hbm.at[s].at[g], dstv.at[gb],
                                      isems.at[gb]).wait()
                pltpu.async_copy(tab_hbm.at[srcv.at[gb].at[0]], rows_v.at[0],
                                 rsems.at[0])

                def chunk(i, carry2):
                    b = lax.rem(i, 2)
                    nb = lax.rem(i + 1, 2)

                    @pl.when(i + 1 < _GRP)
                    def _():
                        pltpu.async_copy(tab_hbm.at[srcv.at[gb].at[i + 1]],
                                         rows_v.at[nb], rsems.at[nb])

                    pltpu.make_async_copy(tab_hbm.at[srcv.at[gb].at[i]],
                                          rows_v.at[b], rsems.at[b]).wait()
                    pltpu.sync_copy(rows_v.at[b],
                                    acc_sh.at[dstv.at[gb].at[i]], add=True)
                    return carry2

                lax.fori_loop(0, _GRP, chunk, 0)
                return carry

            lax.fori_loop(0, ngrp, group, 0)

        @pl.when(c == 0)
        def _():
            run(lo_hbm)

        @pl.when(c == 1)
        def _():
            run(hi_hbm)

        plsc.subcore_barrier()

        @pl.when(c == 0)
        def _():
            pltpu.sync_copy(acc_sh.at[pl.ds(s * npt, npt)], alo_hbm.at[s])

        @pl.when(c == 1)
        def _():
            pltpu.sync_copy(acc_sh.at[pl.ds(s * npt, npt)], ahi_hbm.at[s])

    return spmm_kernel


@functools.cache
def _make_prologue(n: int, d: int, h: int, blk: int):
    def body(x_ref, wt_ref, b_ref, deg_ref,
             x0_ref, lo_ref, hi_ref, dinv_ref):
        hm = jnp.dot(x_ref[...], wt_ref[...],
                     preferred_element_type=jnp.float32) + b_ref[...]
        hm = jnp.maximum(hm, 0.0)
        deg = deg_ref[:, 0:1] + 1.0
        dinv = lax.rsqrt(deg)
        x0_ref[...] = hm
        op = hm * dinv
        lo_ref[...] = op[:, :_CH]
        hi_ref[...] = op[:, _CH:]
        dinv_ref[...] = dinv

    grid = (n // blk,)
    return pl.pallas_call(
        body,
        grid=grid,
        in_specs=[
            pl.BlockSpec((blk, d), lambda i: (i, 0)),
            pl.BlockSpec((d, h), lambda i: (0, 0)),
            pl.BlockSpec((1, h), lambda i: (0, 0)),
            pl.BlockSpec((blk, _CH), lambda i: (i, 0)),
        ],
        out_specs=[
            pl.BlockSpec((blk, h), lambda i: (i, 0)),
            pl.BlockSpec((blk, _CH), lambda i: (i, 0)),
            pl.BlockSpec((blk, _CH), lambda i: (i, 0)),
            pl.BlockSpec((blk, 1), lambda i: (i, 0)),
        ],
        out_shape=[
            jax.ShapeDtypeStruct((n, h), jnp.float32),
            jax.ShapeDtypeStruct((n, _CH), jnp.float32),
            jax.ShapeDtypeStruct((n, _CH), jnp.float32),
            jax.ShapeDtypeStruct((n, 1), jnp.float32),
        ],
    )


@functools.cache
def _make_layer(n: int, h: int, beta: float, last: bool, blk: int):
    def body(alo_ref, ahi_ref, lo_ref, hi_ref, x0_ref, dinv_ref, w_ref, *outs):
        agg = jnp.concatenate(
            [alo_ref[...] + lo_ref[...], ahi_ref[...] + hi_ref[...]], axis=1)
        dinv = dinv_ref[...]
        hc = (1.0 - _ALPHA) * (agg * dinv) + _ALPHA * x0_ref[...]
        out = (1.0 - beta) * hc + beta * jnp.dot(
            hc, w_ref[...], preferred_element_type=jnp.float32)
        if last:
            outs[0][...] = out
        else:
            op = jnp.maximum(out, 0.0) * dinv
            outs[0][...] = op[:, :_CH]
            outs[1][...] = op[:, _CH:]

    grid = (n // blk,)
    in_specs = [
        pl.BlockSpec((blk, _CH), lambda i: (i, 0)),
        pl.BlockSpec((blk, _CH), lambda i: (i, 0)),
        pl.BlockSpec((blk, _CH), lambda i: (i, 0)),
        pl.BlockSpec((blk, _CH), lambda i: (i, 0)),
        pl.BlockSpec((blk, h), lambda i: (i, 0)),
        pl.BlockSpec((blk, 1), lambda i: (i, 0)),
        pl.BlockSpec((h, h), lambda i: (0, 0)),
    ]
    if last:
        out_specs = [pl.BlockSpec((blk, h), lambda i: (i, 0))]
        out_shape = [jax.ShapeDtypeStruct((n, h), jnp.float32)]
    else:
        out_specs = [pl.BlockSpec((blk, _CH), lambda i: (i, 0)),
                     pl.BlockSpec((blk, _CH), lambda i: (i, 0))]
        out_shape = [jax.ShapeDtypeStruct((n, _CH), jnp.float32),
                     jax.ShapeDtypeStruct((n, _CH), jnp.float32)]
    return pl.pallas_call(
        body, grid=grid, in_specs=in_specs, out_specs=out_specs,
        out_shape=out_shape)


def kernel(x, edge_index, lin_w, lin_b, conv_ws):
    n, d = x.shape
    e = edge_index.shape[1]
    h = lin_w.shape[0]
    blk = 1000

    npt = n // 16
    rows_t = e // (16 * _KS)
    src3d = edge_index[0].reshape(16, rows_t // _GRP, _GRP, _KS)
    dst3d = edge_index[1].reshape(16, rows_t // _GRP, _GRP, _KS)
    zeros_c = jnp.zeros((16, npt, _CH), jnp.float32)
    ones_c = jnp.ones((n, _CH), jnp.float32)

    spmm = _make_spmm_kernel(n, e)
    dego, _ = spmm(src3d, dst3d, ones_c, ones_c, zeros_c)
    x0, lo, hi, dinv = _make_prologue(n, d, h, blk)(
        x, lin_w.T, lin_b.reshape(1, h), dego.reshape(n, _CH))
    out = None
    for layer in range(_NUM_LAYERS):
        beta = math.log(_THETA / (layer + 1) + 1.0)
        alo, ahi = spmm(src3d, dst3d, lo, hi, zeros_c)
        last = layer == _NUM_LAYERS - 1
        layer_fn = _make_layer(n, h, beta, last, blk)
        args = (alo.reshape(n, _CH), ahi.reshape(n, _CH), lo, hi, x0, dinv,
                conv_ws[layer])
        if last:
            (out,) = layer_fn(*args)
        else:
            lo, hi = layer_fn(*args)
    return out


# speedup vs baseline: 1.3513x; 1.0094x over previous
import functools
import math

import jax
import jax.numpy as jnp
from jax import lax
from jax.experimental import pallas as pl
from jax.experimental.pallas import tpu as pltpu
from jax.experimental.pallas import tpu_sc as plsc

_ALPHA = 0.2
_THETA = 1.0
_NUM_LAYERS = 8

_KD = 40
_KS = 125
_WD = 16
_CH = 128


_GRP = 20


@functools.cache
def _make_spmm_kernel(n_nodes: int, n_edges: int):
    nrows = n_edges // _KS
    rows_t = nrows // 16
    ngrp = rows_t // _GRP
    npt = n_nodes // 16
    mesh = plsc.VectorSubcoreMesh(core_axis_name="c", subcore_axis_name="s")

    @functools.partial(
        pl.kernel,
        out_type=[jax.ShapeDtypeStruct((16, npt, _CH), jnp.float32),
                  jax.ShapeDtypeStruct((16, npt, _CH), jnp.float32)],
        mesh=mesh,
        scratch_types=[
            pltpu.VMEM((2, _GRP, _KS), jnp.int32),
            pltpu.VMEM((2, _GRP, _KS), jnp.int32),
            pltpu.VMEM((2, _KS, _CH), jnp.float32),
            pltpu.VMEM_SHARED((n_nodes, _CH), jnp.float32),
            pltpu.SemaphoreType.DMA((2,)),
            pltpu.SemaphoreType.DMA((2,)),
            pltpu.SemaphoreType.DMA((2,)),
        ],
    )
    def spmm_kernel(src_hbm, dst_hbm, lo_hbm, hi_hbm, z_hbm,
                    alo_hbm, ahi_hbm, srcv, dstv, rows_v, acc_sh,
                    isems, rsems, ssems):
        c = lax.axis_index("c")
        s = lax.axis_index("s")
        pltpu.sync_copy(z_hbm.at[s], acc_sh.at[pl.ds(s * npt, npt)])
        plsc.subcore_barrier()

        def run(tab_hbm):
            pltpu.async_copy(src_hbm.at[s].at[0], srcv.at[0], isems.at[0])
            pltpu.async_copy(dst_hbm.at[s].at[0], dstv.at[0], isems.at[0])

            def group(g, carry):
                gb = lax.rem(g, 2)
                gnb = lax.rem(g + 1, 2)

                @pl.when(g + 1 < ngrp)
                def _():
                    pltpu.async_copy(src_hbm.at[s].at[g + 1], srcv.at[gnb],
                                     isems.at[gnb])
                    pltpu.async_copy(dst_hbm.at[s].at[g + 1], dstv.at[gnb],
                                     isems.at[gnb])

                pltpu.make_async_copy(src_hbm.at[s].at[g], srcv.at[gb],
                                      isems.at[gb]).wait()
                pltpu.make_async_copy(dst_hbm.at[s].at[g], dstv.at[gb],
                                      isems.at[gb]).wait()
                pltpu.async_copy(tab_hbm.at[srcv.at[gb].at[0]], rows_v.at[0],
                                 rsems.at[0])

                def chunk(i, carry2):
                    b = lax.rem(i, 2)
                    nb = lax.rem(i + 1, 2)

                    @pl.when(i + 1 < _GRP)
                    def _():
                        pltpu.async_copy(tab_hbm.at[srcv.at[gb].at[i + 1]],
                                         rows_v.at[nb], rsems.at[nb])

                    pltpu.make_async_copy(tab_hbm.at[srcv.at[gb].at[i]],
                                          rows_v.at[b], rsems.at[b]).wait()
                    pltpu.sync_copy(rows_v.at[b],
                                    acc_sh.at[dstv.at[gb].at[i]], add=True)
                    return carry2

                lax.fori_loop(0, _GRP, chunk, 0)
                return carry

            lax.fori_loop(0, ngrp, group, 0)

        @pl.when(c == 0)
        def _():
            run(lo_hbm)

        @pl.when(c == 1)
        def _():
            run(hi_hbm)

        plsc.subcore_barrier()

        @pl.when(c == 0)
        def _():
            pltpu.sync_copy(acc_sh.at[pl.ds(s * npt, npt)], alo_hbm.at[s])

        @pl.when(c == 1)
        def _():
            pltpu.sync_copy(acc_sh.at[pl.ds(s * npt, npt)], ahi_hbm.at[s])

    return spmm_kernel


@functools.cache
def _make_prologue(n: int, d: int, h: int, blk: int):
    def body(x_ref, wt_ref, b_ref, deg_ref,
             x0_ref, lo_ref, hi_ref, dinv_ref):
        hm = jnp.dot(x_ref[...], wt_ref[...],
                     preferred_element_type=jnp.float32) + b_ref[...]
        hm = jnp.maximum(hm, 0.0)
        deg = deg_ref[:, 0:1] + 1.0
        dinv = lax.rsqrt(deg)
        x0_ref[...] = hm
        op = hm * dinv
        lo_ref[...] = op[:, :_CH]
        hi_ref[...] = op[:, _CH:]
        dinv_ref[...] = dinv

    grid = (n // blk,)
    return pl.pallas_call(
        body,
        grid=grid,
        in_specs=[
            pl.BlockSpec((blk, d), lambda i: (i, 0)),
            pl.BlockSpec((d, h), lambda i: (0, 0)),
            pl.BlockSpec((1, h), lambda i: (0, 0)),
            pl.BlockSpec((blk, _CH), lambda i: (i, 0)),
        ],
        out_specs=[
            pl.BlockSpec((blk, h), lambda i: (i, 0)),
            pl.BlockSpec((blk, _CH), lambda i: (i, 0)),
            pl.BlockSpec((blk, _CH), lambda i: (i, 0)),
            pl.BlockSpec((blk, 1), lambda i: (i, 0)),
        ],
        out_shape=[
            jax.ShapeDtypeStruct((n, h), jnp.float32),
            jax.ShapeDtypeStruct((n, _CH), jnp.float32),
            jax.ShapeDtypeStruct((n, _CH), jnp.float32),
            jax.ShapeDtypeStruct((n, 1), jnp.float32),
        ],
    )


@functools.cache
def _make_layer(n: int, h: int, beta: float, last: bool, blk: int):
    def body(alo_ref, ahi_ref, lo_ref, hi_ref, x0_ref, dinv_ref, w_ref, *outs):
        agg = jnp.concatenate(
            [alo_ref[...] + lo_ref[...], ahi_ref[...] + hi_ref[...]], axis=1)
        dinv = dinv_ref[...]
        hc = (1.0 - _ALPHA) * (agg * dinv) + _ALPHA * x0_ref[...]
        out = (1.0 - beta) * hc + beta * jnp.dot(
            hc, w_ref[...], preferred_element_type=jnp.float32)
        if last:
            outs[0][...] = out
        else:
            op = jnp.maximum(out, 0.0) * dinv
            outs[0][...] = op[:, :_CH]
            outs[1][...] = op[:, _CH:]

    grid = (n // blk,)
    in_specs = [
        pl.BlockSpec((blk, _CH), lambda i: (i, 0)),
        pl.BlockSpec((blk, _CH), lambda i: (i, 0)),
        pl.BlockSpec((blk, _CH), lambda i: (i, 0)),
        pl.BlockSpec((blk, _CH), lambda i: (i, 0)),
        pl.BlockSpec((blk, h), lambda i: (i, 0)),
        pl.BlockSpec((blk, 1), lambda i: (i, 0)),
        pl.BlockSpec((h, h), lambda i: (0, 0)),
    ]
    if last:
        out_specs = [pl.BlockSpec((blk, h), lambda i: (i, 0))]
        out_shape = [jax.ShapeDtypeStruct((n, h), jnp.float32)]
    else:
        out_specs = [pl.BlockSpec((blk, _CH), lambda i: (i, 0)),
                     pl.BlockSpec((blk, _CH), lambda i: (i, 0))]
        out_shape = [jax.ShapeDtypeStruct((n, _CH), jnp.float32),
                     jax.ShapeDtypeStruct((n, _CH), jnp.float32)]
    return pl.pallas_call(
        body, grid=grid, in_specs=in_specs, out_specs=out_specs,
        out_shape=out_shape)


def kernel(x, edge_index, lin_w, lin_b, conv_ws):
    n, d = x.shape
    e = edge_index.shape[1]
    h = lin_w.shape[0]
    blk = 2000

    npt = n // 16
    rows_t = e // (16 * _KS)
    src3d = edge_index[0].reshape(16, rows_t // _GRP, _GRP, _KS)
    dst3d = edge_index[1].reshape(16, rows_t // _GRP, _GRP, _KS)
    zeros_c = jnp.zeros((16, npt, _CH), jnp.float32)
    ones_c = jnp.ones((n, _CH), jnp.float32)

    spmm = _make_spmm_kernel(n, e)
    dego, _ = spmm(src3d, dst3d, ones_c, ones_c, zeros_c)
    x0, lo, hi, dinv = _make_prologue(n, d, h, blk)(
        x, lin_w.T, lin_b.reshape(1, h), dego.reshape(n, _CH))
    out = None
    for layer in range(_NUM_LAYERS):
        beta = math.log(_THETA / (layer + 1) + 1.0)
        alo, ahi = spmm(src3d, dst3d, lo, hi, zeros_c)
        last = layer == _NUM_LAYERS - 1
        layer_fn = _make_layer(n, h, beta, last, blk)
        args = (alo.reshape(n, _CH), ahi.reshape(n, _CH), lo, hi, x0, dinv,
                conv_ws[layer])
        if last:
            (out,) = layer_fn(*args)
        else:
            lo, hi = layer_fn(*args)
    return out


# speedup vs baseline: 1.4303x; 1.0585x over previous
import functools
import math

import jax
import jax.numpy as jnp
from jax import lax
from jax.experimental import pallas as pl
from jax.experimental.pallas import tpu as pltpu
from jax.experimental.pallas import tpu_sc as plsc

_ALPHA = 0.2
_THETA = 1.0
_NUM_LAYERS = 8

_KD = 40
_KS = 125
_WD = 16
_CH = 128


@functools.cache
def _make_deg_kernel(n_nodes: int, n_edges: int):
    rows_w = n_edges // (32 * _KS)
    npt = n_nodes // 16
    mesh = plsc.VectorSubcoreMesh(core_axis_name="c", subcore_axis_name="s")

    @functools.partial(
        pl.kernel,
        out_type=[jax.ShapeDtypeStruct((16, npt, _CH), jnp.float32),
                  jax.ShapeDtypeStruct((16, npt, _CH), jnp.float32)],
        mesh=mesh,
        scratch_types=[
            pltpu.VMEM((rows_w, _KS), jnp.int32),
            pltpu.VMEM((_KS, _CH), jnp.float32),
            pltpu.VMEM_SHARED((n_nodes, _CH), jnp.float32),
        ],
    )
    def deg_kernel(dst_hbm, z_hbm, ones_hbm, dega_hbm, degb_hbm,
                   idxv, ones_v, hist_sh):
        c = lax.axis_index("c")
        s = lax.axis_index("s")
        pltpu.sync_copy(z_hbm.at[s], hist_sh.at[pl.ds(s * npt, npt)])
        plsc.subcore_barrier()
        pltpu.sync_copy(dst_hbm.at[c * 16 + s], idxv)
        pltpu.sync_copy(ones_hbm.at[idxv.at[0]], ones_v)

        def chunk(i, carry):
            pltpu.sync_copy(ones_v, hist_sh.at[idxv.at[i]], add=True)
            return carry

        lax.fori_loop(0, rows_w, chunk, 0)
        plsc.subcore_barrier()

        @pl.when(c == 0)
        def _():
            pltpu.sync_copy(hist_sh.at[pl.ds(s * npt, npt)], dega_hbm.at[s])

        @pl.when(c == 1)
        def _():
            pltpu.sync_copy(hist_sh.at[pl.ds(s * npt, npt)], degb_hbm.at[s])

    return deg_kernel


_GRP = 20


@functools.cache
def _make_spmm_kernel(n_nodes: int, n_edges: int):
    nrows = n_edges // _KS
    rows_t = nrows // 16
    ngrp = rows_t // _GRP
    npt = n_nodes // 16
    mesh = plsc.VectorSubcoreMesh(core_axis_name="c", subcore_axis_name="s")

    @functools.partial(
        pl.kernel,
        out_type=[jax.ShapeDtypeStruct((16, npt, _CH), jnp.float32),
                  jax.ShapeDtypeStruct((16, npt, _CH), jnp.float32)],
        mesh=mesh,
        scratch_types=[
            pltpu.VMEM((2, _GRP, _KS), jnp.int32),
            pltpu.VMEM((2, _GRP, _KS), jnp.int32),
            pltpu.VMEM((2, _KS, _CH), jnp.float32),
            pltpu.VMEM_SHARED((n_nodes, _CH), jnp.float32),
            pltpu.SemaphoreType.DMA((2,)),
            pltpu.SemaphoreType.DMA((2,)),
            pltpu.SemaphoreType.DMA((2,)),
        ],
    )
    def spmm_kernel(src_hbm, dst_hbm, lo_hbm, hi_hbm, z_hbm,
                    alo_hbm, ahi_hbm, srcv, dstv, rows_v, acc_sh,
                    isems, rsems, ssems):
        c = lax.axis_index("c")
        s = lax.axis_index("s")
        pltpu.sync_copy(z_hbm.at[s], acc_sh.at[pl.ds(s * npt, npt)])
        plsc.subcore_barrier()

        def run(tab_hbm):
            pltpu.async_copy(src_hbm.at[s].at[0], srcv.at[0], isems.at[0])
            pltpu.async_copy(dst_hbm.at[s].at[0], dstv.at[0], isems.at[0])

            def group(g, carry):
                gb = lax.rem(g, 2)
                gnb = lax.rem(g + 1, 2)

                @pl.when(g + 1 < ngrp)
                def _():
                    pltpu.async_copy(src_hbm.at[s].at[g + 1], srcv.at[gnb],
                                     isems.at[gnb])
                    pltpu.async_copy(dst_hbm.at[s].at[g + 1], dstv.at[gnb],
                                     isems.at[gnb])

                pltpu.make_async_copy(src_hbm.at[s].at[g], srcv.at[gb],
                                      isems.at[gb]).wait()
                pltpu.make_async_copy(dst_hbm.at[s].at[g], dstv.at[gb],
                                      isems.at[gb]).wait()
                pltpu.async_copy(tab_hbm.at[srcv.at[gb].at[0]], rows_v.at[0],
                                 rsems.at[0])

                def chunk(i, carry2):
                    b = lax.rem(i, 2)
                    nb = lax.rem(i + 1, 2)

                    @pl.when(i + 1 < _GRP)
                    def _():
                        pltpu.async_copy(tab_hbm.at[srcv.at[gb].at[i + 1]],
                                         rows_v.at[nb], rsems.at[nb])

                    pltpu.make_async_copy(tab_hbm.at[srcv.at[gb].at[i]],
                                          rows_v.at[b], rsems.at[b]).wait()
                    pltpu.sync_copy(rows_v.at[b],
                                    acc_sh.at[dstv.at[gb].at[i]], add=True)
                    return carry2

                lax.fori_loop(0, _GRP, chunk, 0)
                return carry

            lax.fori_loop(0, ngrp, group, 0)

        @pl.when(c == 0)
        def _():
            run(lo_hbm)

        @pl.when(c == 1)
        def _():
            run(hi_hbm)

        plsc.subcore_barrier()

        @pl.when(c == 0)
        def _():
            pltpu.sync_copy(acc_sh.at[pl.ds(s * npt, npt)], alo_hbm.at[s])

        @pl.when(c == 1)
        def _():
            pltpu.sync_copy(acc_sh.at[pl.ds(s * npt, npt)], ahi_hbm.at[s])

    return spmm_kernel


@functools.cache
def _make_prologue(n: int, d: int, h: int, blk: int):
    def body(x_ref, wt_ref, b_ref, dega_ref, degb_ref,
             x0_ref, lo_ref, hi_ref, dinv_ref):
        hm = jnp.dot(x_ref[...], wt_ref[...],
                     preferred_element_type=jnp.float32) + b_ref[...]
        hm = jnp.maximum(hm, 0.0)
        deg = dega_ref[:, 0:1] + degb_ref[:, 0:1] + 1.0
        dinv = lax.rsqrt(deg)
        x0_ref[...] = hm
        op = hm * dinv
        lo_ref[...] = op[:, :_CH]
        hi_ref[...] = op[:, _CH:]
        dinv_ref[...] = dinv

    grid = (n // blk,)
    return pl.pallas_call(
        body,
        grid=grid,
        in_specs=[
            pl.BlockSpec((blk, d), lambda i: (i, 0)),
            pl.BlockSpec((d, h), lambda i: (0, 0)),
            pl.BlockSpec((1, h), lambda i: (0, 0)),
            pl.BlockSpec((blk, _CH), lambda i: (i, 0)),
            pl.BlockSpec((blk, _CH), lambda i: (i, 0)),
        ],
        out_specs=[
            pl.BlockSpec((blk, h), lambda i: (i, 0)),
            pl.BlockSpec((blk, _CH), lambda i: (i, 0)),
            pl.BlockSpec((blk, _CH), lambda i: (i, 0)),
            pl.BlockSpec((blk, 1), lambda i: (i, 0)),
        ],
        out_shape=[
            jax.ShapeDtypeStruct((n, h), jnp.float32),
            jax.ShapeDtypeStruct((n, _CH), jnp.float32),
            jax.ShapeDtypeStruct((n, _CH), jnp.float32),
            jax.ShapeDtypeStruct((n, 1), jnp.float32),
        ],
    )


@functools.cache
def _make_layer(n: int, h: int, beta: float, last: bool, blk: int):
    def body(alo_ref, ahi_ref, lo_ref, hi_ref, x0_ref, dinv_ref, w_ref, *outs):
        agg = jnp.concatenate(
            [alo_ref[...] + lo_ref[...], ahi_ref[...] + hi_ref[...]], axis=1)
        dinv = dinv_ref[...]
        hc = (1.0 - _ALPHA) * (agg * dinv) + _ALPHA * x0_ref[...]
        out = (1.0 - beta) * hc + beta * jnp.dot(
            hc, w_ref[...], preferred_element_type=jnp.float32)
        if last:
            outs[0][...] = out
        else:
            op = jnp.maximum(out, 0.0) * dinv
            outs[0][...] = op[:, :_CH]
            outs[1][...] = op[:, _CH:]

    grid = (n // blk,)
    in_specs = [
        pl.BlockSpec((blk, _CH), lambda i: (i, 0)),
        pl.BlockSpec((blk, _CH), lambda i: (i, 0)),
        pl.BlockSpec((blk, _CH), lambda i: (i, 0)),
        pl.BlockSpec((blk, _CH), lambda i: (i, 0)),
        pl.BlockSpec((blk, h), lambda i: (i, 0)),
        pl.BlockSpec((blk, 1), lambda i: (i, 0)),
        pl.BlockSpec((h, h), lambda i: (0, 0)),
    ]
    if last:
        out_specs = [pl.BlockSpec((blk, h), lambda i: (i, 0))]
        out_shape = [jax.ShapeDtypeStruct((n, h), jnp.float32)]
    else:
        out_specs = [pl.BlockSpec((blk, _CH), lambda i: (i, 0)),
                     pl.BlockSpec((blk, _CH), lambda i: (i, 0))]
        out_shape = [jax.ShapeDtypeStruct((n, _CH), jnp.float32),
                     jax.ShapeDtypeStruct((n, _CH), jnp.float32)]
    return pl.pallas_call(
        body, grid=grid, in_specs=in_specs, out_specs=out_specs,
        out_shape=out_shape)


def kernel(x, edge_index, lin_w, lin_b, conv_ws):
    n, d = x.shape
    e = edge_index.shape[1]
    h = lin_w.shape[0]
    blk = 2000

    npt = n // 16
    rows_t = e // (16 * _KS)
    src3d = edge_index[0].reshape(16, rows_t // _GRP, _GRP, _KS)
    dst3d = edge_index[1].reshape(16, rows_t // _GRP, _GRP, _KS)
    zeros_c = jnp.zeros((16, npt, _CH), jnp.float32)
    dst_deg = edge_index[1].reshape(32, e // (32 * _KS), _KS)
    ones_c = jnp.ones((n, _CH), jnp.float32)

    spmm = _make_spmm_kernel(n, e)
    dega, degb = _make_deg_kernel(n, e)(dst_deg, zeros_c, ones_c)
    x0, lo, hi, dinv = _make_prologue(n, d, h, blk)(
        x, lin_w.T, lin_b.reshape(1, h),
        dega.reshape(n, _CH), degb.reshape(n, _CH))
    out = None
    for layer in range(_NUM_LAYERS):
        beta = math.log(_THETA / (layer + 1) + 1.0)
        alo, ahi = spmm(src3d, dst3d, lo, hi, zeros_c)
        last = layer == _NUM_LAYERS - 1
        layer_fn = _make_layer(n, h, beta, last, blk)
        args = (alo.reshape(n, _CH), ahi.reshape(n, _CH), lo, hi, x0, dinv,
                conv_ws[layer])
        if last:
            (out,) = layer_fn(*args)
        else:
            lo, hi = layer_fn(*args)
    return out


# speedup vs baseline: 1.4374x; 1.0049x over previous
import functools
import math

import jax
import jax.numpy as jnp
from jax import lax
from jax.experimental import pallas as pl
from jax.experimental.pallas import tpu as pltpu
from jax.experimental.pallas import tpu_sc as plsc

_ALPHA = 0.2
_THETA = 1.0
_NUM_LAYERS = 8

_KD = 40
_KS = 125
_WD = 16
_CH = 128


@functools.cache
def _make_deg_kernel(n_nodes: int, n_edges: int):
    rows_w = n_edges // (32 * _KS)
    npt = n_nodes // 16
    mesh = plsc.VectorSubcoreMesh(core_axis_name="c", subcore_axis_name="s")

    @functools.partial(
        pl.kernel,
        out_type=[jax.ShapeDtypeStruct((16, npt, _CH), jnp.float32),
                  jax.ShapeDtypeStruct((16, npt, _CH), jnp.float32)],
        mesh=mesh,
        scratch_types=[
            pltpu.VMEM((rows_w, _KS), jnp.int32),
            pltpu.VMEM((_KS, _CH), jnp.float32),
            pltpu.VMEM_SHARED((n_nodes, _CH), jnp.float32),
        ],
    )
    def deg_kernel(dst_hbm, z_hbm, ones_hbm, dega_hbm, degb_hbm,
                   idxv, ones_v, hist_sh):
        c = lax.axis_index("c")
        s = lax.axis_index("s")
        pltpu.sync_copy(z_hbm.at[s], hist_sh.at[pl.ds(s * npt, npt)])
        plsc.subcore_barrier()
        pltpu.sync_copy(dst_hbm.at[c * 16 + s], idxv)
        pltpu.sync_copy(ones_hbm.at[idxv.at[0]], ones_v)

        def chunk(i, carry):
            pltpu.sync_copy(ones_v, hist_sh.at[idxv.at[i]], add=True)
            return carry

        lax.fori_loop(0, rows_w, chunk, 0)
        plsc.subcore_barrier()

        @pl.when(c == 0)
        def _():
            pltpu.sync_copy(hist_sh.at[pl.ds(s * npt, npt)], dega_hbm.at[s])

        @pl.when(c == 1)
        def _():
            pltpu.sync_copy(hist_sh.at[pl.ds(s * npt, npt)], degb_hbm.at[s])

    return deg_kernel


_GRP = 20


@functools.cache
def _make_spmm_kernel(n_nodes: int, n_edges: int):
    nrows = n_edges // _KS
    rows_t = nrows // 16
    ngrp = rows_t // _GRP
    npt = n_nodes // 16
    mesh = plsc.VectorSubcoreMesh(core_axis_name="c", subcore_axis_name="s")

    @functools.partial(
        pl.kernel,
        out_type=[jax.ShapeDtypeStruct((16, npt, _CH), jnp.float32),
                  jax.ShapeDtypeStruct((16, npt, _CH), jnp.float32)],
        mesh=mesh,
        scratch_types=[
            pltpu.VMEM((2, _GRP, _KS), jnp.int32),
            pltpu.VMEM((2, _GRP, _KS), jnp.int32),
            pltpu.VMEM((2, _KS, _CH), jnp.float32),
            pltpu.VMEM_SHARED((n_nodes, _CH), jnp.float32),
            pltpu.SemaphoreType.DMA((2,)),
            pltpu.SemaphoreType.DMA((2,)),
            pltpu.SemaphoreType.DMA((2,)),
        ],
    )
    def spmm_kernel(src_hbm, dst_hbm, lo_hbm, hi_hbm, z_hbm,
                    alo_hbm, ahi_hbm, srcv, dstv, rows_v, acc_sh,
                    isems, rsems, ssems):
        c = lax.axis_index("c")
        s = lax.axis_index("s")
        pltpu.async_copy(src_hbm.at[s].at[0], srcv.at[0], isems.at[0])
        pltpu.async_copy(dst_hbm.at[s].at[0], dstv.at[0], isems.at[0])
        pltpu.sync_copy(z_hbm.at[s], acc_sh.at[pl.ds(s * npt, npt)])
        plsc.subcore_barrier()

        def run(tab_hbm):
            def group(g, carry):
                gb = lax.rem(g, 2)
                gnb = lax.rem(g + 1, 2)

                @pl.when(g + 1 < ngrp)
                def _():
                    pltpu.async_copy(src_hbm.at[s].at[g + 1], srcv.at[gnb],
                                     isems.at[gnb])
                    pltpu.async_copy(dst_hbm.at[s].at[g + 1], dstv.at[gnb],
                                     isems.at[gnb])

                pltpu.make_async_copy(src_hbm.at[s].at[g], srcv.at[gb],
                                      isems.at[gb]).wait()
                pltpu.make_async_copy(dst_hbm.at[s].at[g], dstv.at[gb],
                                      isems.at[gb]).wait()
                pltpu.async_copy(tab_hbm.at[srcv.at[gb].at[0]], rows_v.at[0],
                                 rsems.at[0])

                def chunk(i, carry2):
                    b = lax.rem(i, 2)
                    nb = lax.rem(i + 1, 2)

                    @pl.when(i + 1 < _GRP)
                    def _():
                        pltpu.async_copy(tab_hbm.at[srcv.at[gb].at[i + 1]],
                                         rows_v.at[nb], rsems.at[nb])

                    pltpu.make_async_copy(tab_hbm.at[srcv.at[gb].at[i]],
                                          rows_v.at[b], rsems.at[b]).wait()
                    pltpu.sync_copy(rows_v.at[b],
                                    acc_sh.at[dstv.at[gb].at[i]], add=True)
                    return carry2

                lax.fori_loop(0, _GRP, chunk, 0)
                return carry

            lax.fori_loop(0, ngrp, group, 0)

        @pl.when(c == 0)
        def _():
            run(lo_hbm)

        @pl.when(c == 1)
        def _():
            run(hi_hbm)

        plsc.subcore_barrier()

        @pl.when(c == 0)
        def _():
            pltpu.sync_copy(acc_sh.at[pl.ds(s * npt, npt)], alo_hbm.at[s])

        @pl.when(c == 1)
        def _():
            pltpu.sync_copy(acc_sh.at[pl.ds(s * npt, npt)], ahi_hbm.at[s])

    return spmm_kernel


@functools.cache
def _make_prologue(n: int, d: int, h: int, blk: int):
    def body(x_ref, wt_ref, b_ref, dega_ref, degb_ref,
             x0_ref, lo_ref, hi_ref, dinv_ref):
        hm = jnp.dot(x_ref[...], wt_ref[...],
                     preferred_element_type=jnp.float32) + b_ref[...]
        hm = jnp.maximum(hm, 0.0)
        deg = dega_ref[:, 0:1] + degb_ref[:, 0:1] + 1.0
        dinv = lax.rsqrt(deg)
        x0_ref[...] = hm
        op = hm * dinv
        lo_ref[...] = op[:, :_CH]
        hi_ref[...] = op[:, _CH:]
        dinv_ref[...] = dinv

    grid = (n // blk,)
    return pl.pallas_call(
        body,
        grid=grid,
        in_specs=[
            pl.BlockSpec((blk, d), lambda i: (i, 0)),
            pl.BlockSpec((d, h), lambda i: (0, 0)),
            pl.BlockSpec((1, h), lambda i: (0, 0)),
            pl.BlockSpec((blk, _CH), lambda i: (i, 0)),
            pl.BlockSpec((blk, _CH), lambda i: (i, 0)),
        ],
        out_specs=[
            pl.BlockSpec((blk, h), lambda i: (i, 0)),
            pl.BlockSpec((blk, _CH), lambda i: (i, 0)),
            pl.BlockSpec((blk, _CH), lambda i: (i, 0)),
            pl.BlockSpec((blk, 1), lambda i: (i, 0)),
        ],
        out_shape=[
            jax.ShapeDtypeStruct((n, h), jnp.float32),
            jax.ShapeDtypeStruct((n, _CH), jnp.float32),
            jax.ShapeDtypeStruct((n, _CH), jnp.float32),
            jax.ShapeDtypeStruct((n, 1), jnp.float32),
        ],
    )


@functools.cache
def _make_layer(n: int, h: int, beta: float, last: bool, blk: int):
    def body(alo_ref, ahi_ref, lo_ref, hi_ref, x0_ref, dinv_ref, w_ref, *outs):
        agg = jnp.concatenate(
            [alo_ref[...] + lo_ref[...], ahi_ref[...] + hi_ref[...]], axis=1)
        dinv = dinv_ref[...]
        hc = (1.0 - _ALPHA) * (agg * dinv) + _ALPHA * x0_ref[...]
        out = (1.0 - beta) * hc + beta * jnp.dot(
            hc, w_ref[...], preferred_element_type=jnp.float32)
        if last:
            outs[0][...] = out
        else:
            op = jnp.maximum(out, 0.0) * dinv
            outs[0][...] = op[:, :_CH]
            outs[1][...] = op[:, _CH:]

    grid = (n // blk,)
    in_specs = [
        pl.BlockSpec((blk, _CH), lambda i: (i, 0)),
        pl.BlockSpec((blk, _CH), lambda i: (i, 0)),
        pl.BlockSpec((blk, _CH), lambda i: (i, 0)),
        pl.BlockSpec((blk, _CH), lambda i: (i, 0)),
        pl.BlockSpec((blk, h), lambda i: (i, 0)),
        pl.BlockSpec((blk, 1), lambda i: (i, 0)),
        pl.BlockSpec((h, h), lambda i: (0, 0)),
    ]
    if last:
        out_specs = [pl.BlockSpec((blk, h), lambda i: (i, 0))]
        out_shape = [jax.ShapeDtypeStruct((n, h), jnp.float32)]
    else:
        out_specs = [pl.BlockSpec((blk, _CH), lambda i: (i, 0)),
                     pl.BlockSpec((blk, _CH), lambda i: (i, 0))]
        out_shape = [jax.ShapeDtypeStruct((n, _CH), jnp.float32),
                     jax.ShapeDtypeStruct((n, _CH), jnp.float32)]
    return pl.pallas_call(
        body, grid=grid, in_specs=in_specs, out_specs=out_specs,
        out_shape=out_shape)


def kernel(x, edge_index, lin_w, lin_b, conv_ws):
    n, d = x.shape
    e = edge_index.shape[1]
    h = lin_w.shape[0]
    blk = 2000

    npt = n // 16
    rows_t = e // (16 * _KS)
    src3d = edge_index[0].reshape(16, rows_t // _GRP, _GRP, _KS)
    dst3d = edge_index[1].reshape(16, rows_t // _GRP, _GRP, _KS)
    zeros_c = jnp.zeros((16, npt, _CH), jnp.float32)
    dst_deg = edge_index[1].reshape(32, e // (32 * _KS), _KS)
    ones_c = jnp.ones((n, _CH), jnp.float32)

    spmm = _make_spmm_kernel(n, e)
    dega, degb = _make_deg_kernel(n, e)(dst_deg, zeros_c, ones_c)
    x0, lo, hi, dinv = _make_prologue(n, d, h, blk)(
        x, lin_w.T, lin_b.reshape(1, h),
        dega.reshape(n, _CH), degb.reshape(n, _CH))
    out = None
    for layer in range(_NUM_LAYERS):
        beta = math.log(_THETA / (layer + 1) + 1.0)
        alo, ahi = spmm(src3d, dst3d, lo, hi, zeros_c)
        last = layer == _NUM_LAYERS - 1
        layer_fn = _make_layer(n, h, beta, last, blk)
        args = (alo.reshape(n, _CH), ahi.reshape(n, _CH), lo, hi, x0, dinv,
                conv_ws[layer])
        if last:
            (out,) = layer_fn(*args)
        else:
            lo, hi = layer_fn(*args)
    return out


# speedup vs baseline: 1.5234x; 1.0598x over previous
import functools
import math

import jax
import jax.numpy as jnp
from jax import lax
from jax.experimental import pallas as pl
from jax.experimental.pallas import tpu as pltpu
from jax.experimental.pallas import tpu_sc as plsc

_ALPHA = 0.2
_THETA = 1.0
_NUM_LAYERS = 8

_KD = 40
_KS = 125
_WD = 16
_CH = 128


@functools.cache
def _make_deg_kernel(n_nodes: int, n_edges: int):
    rows_w = n_edges // (32 * _KS)
    npt = n_nodes // 16
    mesh = plsc.VectorSubcoreMesh(core_axis_name="c", subcore_axis_name="s")

    @functools.partial(
        pl.kernel,
        out_type=[jax.ShapeDtypeStruct((n_nodes, _CH), jnp.float32),
                  jax.ShapeDtypeStruct((n_nodes, _CH), jnp.float32)],
        mesh=mesh,
        scratch_types=[
            pltpu.VMEM((rows_w, _KS), jnp.int32),
            pltpu.VMEM((_KS, _CH), jnp.float32),
            pltpu.VMEM_SHARED((n_nodes, _CH), jnp.float32),
        ],
    )
    def deg_kernel(dst_hbm, z_hbm, ones_hbm, dega_hbm, degb_hbm,
                   idxv, ones_v, hist_sh):
        c = lax.axis_index("c")
        s = lax.axis_index("s")
        ali = -(-npt // 8) * 8
        tail = n_nodes - 15 * ali

        @pl.when(s < 15)
        def _():
            pltpu.sync_copy(z_hbm.at[pl.ds(s * ali, ali)],
                            hist_sh.at[pl.ds(s * ali, ali)])

        @pl.when(s == 15)
        def _():
            pltpu.sync_copy(z_hbm.at[pl.ds(15 * ali, tail)],
                            hist_sh.at[pl.ds(15 * ali, tail)])

        plsc.subcore_barrier()
        pltpu.sync_copy(dst_hbm.at[c * 16 + s], idxv)
        pltpu.sync_copy(ones_hbm.at[idxv.at[0]], ones_v)

        def chunk(i, carry):
            pltpu.sync_copy(ones_v, hist_sh.at[idxv.at[i]], add=True)
            return carry

        lax.fori_loop(0, rows_w, chunk, 0)
        plsc.subcore_barrier()

        def writeout(out_hbm):
            @pl.when(s < 15)
            def _():
                pltpu.sync_copy(hist_sh.at[pl.ds(s * ali, ali)],
                                out_hbm.at[pl.ds(s * ali, ali)])

            @pl.when(s == 15)
            def _():
                pltpu.sync_copy(hist_sh.at[pl.ds(15 * ali, tail)],
                                out_hbm.at[pl.ds(15 * ali, tail)])

        @pl.when(c == 0)
        def _():
            writeout(dega_hbm)

        @pl.when(c == 1)
        def _():
            writeout(degb_hbm)

    return deg_kernel


_GRP = 20


@functools.cache
def _make_spmm_kernel(n_nodes: int, n_edges: int):
    nrows = n_edges // _KS
    rows_t = nrows // 16
    ngrp = rows_t // _GRP
    npt = n_nodes // 16
    mesh = plsc.VectorSubcoreMesh(core_axis_name="c", subcore_axis_name="s")

    @functools.partial(
        pl.kernel,
        out_type=[jax.ShapeDtypeStruct((n_nodes, _CH), jnp.float32),
                  jax.ShapeDtypeStruct((n_nodes, _CH), jnp.float32)],
        mesh=mesh,
        scratch_types=[
            pltpu.VMEM((2, _GRP, _KS), jnp.int32),
            pltpu.VMEM((2, _GRP, _KS), jnp.int32),
            pltpu.VMEM((2, _KS, _CH), jnp.float32),
            pltpu.VMEM_SHARED((n_nodes, _CH), jnp.float32),
            pltpu.SemaphoreType.DMA((2,)),
            pltpu.SemaphoreType.DMA((2,)),
            pltpu.SemaphoreType.DMA((2,)),
        ],
    )
    def spmm_kernel(src_hbm, dst_hbm, lo_hbm, hi_hbm, z_hbm,
                    alo_hbm, ahi_hbm, srcv, dstv, rows_v, acc_sh,
                    isems, rsems, ssems):
        c = lax.axis_index("c")
        s = lax.axis_index("s")
        pltpu.async_copy(src_hbm.at[s].at[0], srcv.at[0], isems.at[0])
        pltpu.async_copy(dst_hbm.at[s].at[0], dstv.at[0], isems.at[0])
        ali = -(-npt // 8) * 8
        tail = n_nodes - 15 * ali

        @pl.when(s < 15)
        def _():
            pltpu.sync_copy(z_hbm.at[pl.ds(s * ali, ali)],
                            acc_sh.at[pl.ds(s * ali, ali)])

        @pl.when(s == 15)
        def _():
            pltpu.sync_copy(z_hbm.at[pl.ds(15 * ali, tail)],
                            acc_sh.at[pl.ds(15 * ali, tail)])

        plsc.subcore_barrier()

        def run(tab_hbm):
            def group(g, carry):
                gb = lax.rem(g, 2)
                gnb = lax.rem(g + 1, 2)

                @pl.when(g + 1 < ngrp)
                def _():
                    pltpu.async_copy(src_hbm.at[s].at[g + 1], srcv.at[gnb],
                                     isems.at[gnb])
                    pltpu.async_copy(dst_hbm.at[s].at[g + 1], dstv.at[gnb],
                                     isems.at[gnb])

                pltpu.make_async_copy(src_hbm.at[s].at[g], srcv.at[gb],
                                      isems.at[gb]).wait()
                pltpu.make_async_copy(dst_hbm.at[s].at[g], dstv.at[gb],
                                      isems.at[gb]).wait()
                pltpu.async_copy(tab_hbm.at[srcv.at[gb].at[0]], rows_v.at[0],
                                 rsems.at[0])

                def chunk(i, carry2):
                    b = lax.rem(i, 2)
                    nb = lax.rem(i + 1, 2)

                    @pl.when(i + 1 < _GRP)
                    def _():
                        pltpu.async_copy(tab_hbm.at[srcv.at[gb].at[i + 1]],
                                         rows_v.at[nb], rsems.at[nb])

                    pltpu.make_async_copy(tab_hbm.at[srcv.at[gb].at[i]],
                                          rows_v.at[b], rsems.at[b]).wait()
                    pltpu.sync_copy(rows_v.at[b],
                                    acc_sh.at[dstv.at[gb].at[i]], add=True)
                    return carry2

                lax.fori_loop(0, _GRP, chunk, 0)
                return carry

            lax.fori_loop(0, ngrp, group, 0)

        @pl.when(c == 0)
        def _():
            run(lo_hbm)

        @pl.when(c == 1)
        def _():
            run(hi_hbm)

        plsc.subcore_barrier()

        def writeout(dst_hbm):
            @pl.when(s < 15)
            def _():
                pltpu.sync_copy(acc_sh.at[pl.ds(s * ali, ali)],
                                dst_hbm.at[pl.ds(s * ali, ali)])

            @pl.when(s == 15)
            def _():
                pltpu.sync_copy(acc_sh.at[pl.ds(15 * ali, tail)],
                                dst_hbm.at[pl.ds(15 * ali, tail)])

        @pl.when(c == 0)
        def _():
            writeout(alo_hbm)

        @pl.when(c == 1)
        def _():
            writeout(ahi_hbm)

    return spmm_kernel


@functools.cache
def _make_prologue(n: int, d: int, h: int, blk: int):
    def body(x_ref, wt_ref, b_ref, dega_ref, degb_ref,
             x0_ref, lo_ref, hi_ref, dinv_ref):
        hm = jnp.dot(x_ref[...], wt_ref[...],
                     preferred_element_type=jnp.float32) + b_ref[...]
        hm = jnp.maximum(hm, 0.0)
        deg = dega_ref[:, 0:1] + degb_ref[:, 0:1] + 1.0
        dinv = lax.rsqrt(deg)
        x0_ref[...] = hm
        op = hm * dinv
        lo_ref[...] = op[:, :_CH]
        hi_ref[...] = op[:, _CH:]
        dinv_ref[...] = dinv

    grid = (n // blk,)
    return pl.pallas_call(
        body,
        grid=grid,
        in_specs=[
            pl.BlockSpec((blk, d), lambda i: (i, 0)),
            pl.BlockSpec((d, h), lambda i: (0, 0)),
            pl.BlockSpec((1, h), lambda i: (0, 0)),
            pl.BlockSpec((blk, _CH), lambda i: (i, 0)),
            pl.BlockSpec((blk, _CH), lambda i: (i, 0)),
        ],
        out_specs=[
            pl.BlockSpec((blk, h), lambda i: (i, 0)),
            pl.BlockSpec((blk, _CH), lambda i: (i, 0)),
            pl.BlockSpec((blk, _CH), lambda i: (i, 0)),
            pl.BlockSpec((blk, 1), lambda i: (i, 0)),
        ],
        out_shape=[
            jax.ShapeDtypeStruct((n, h), jnp.float32),
            jax.ShapeDtypeStruct((n, _CH), jnp.float32),
            jax.ShapeDtypeStruct((n, _CH), jnp.float32),
            jax.ShapeDtypeStruct((n, 1), jnp.float32),
        ],
    )


@functools.cache
def _make_layer(n: int, h: int, beta: float, last: bool, blk: int):
    def body(alo_ref, ahi_ref, lo_ref, hi_ref, x0_ref, dinv_ref, w_ref, *outs):
        agg = jnp.concatenate(
            [alo_ref[...] + lo_ref[...], ahi_ref[...] + hi_ref[...]], axis=1)
        dinv = dinv_ref[...]
        hc = (1.0 - _ALPHA) * (agg * dinv) + _ALPHA * x0_ref[...]
        out = (1.0 - beta) * hc + beta * jnp.dot(
            hc, w_ref[...], preferred_element_type=jnp.float32)
        if last:
            outs[0][...] = out
        else:
            op = jnp.maximum(out, 0.0) * dinv
            outs[0][...] = op[:, :_CH]
            outs[1][...] = op[:, _CH:]

    grid = (n // blk,)
    in_specs = [
        pl.BlockSpec((blk, _CH), lambda i: (i, 0)),
        pl.BlockSpec((blk, _CH), lambda i: (i, 0)),
        pl.BlockSpec((blk, _CH), lambda i: (i, 0)),
        pl.BlockSpec((blk, _CH), lambda i: (i, 0)),
        pl.BlockSpec((blk, h), lambda i: (i, 0)),
        pl.BlockSpec((blk, 1), lambda i: (i, 0)),
        pl.BlockSpec((h, h), lambda i: (0, 0)),
    ]
    if last:
        out_specs = [pl.BlockSpec((blk, h), lambda i: (i, 0))]
        out_shape = [jax.ShapeDtypeStruct((n, h), jnp.float32)]
    else:
        out_specs = [pl.BlockSpec((blk, _CH), lambda i: (i, 0)),
                     pl.BlockSpec((blk, _CH), lambda i: (i, 0))]
        out_shape = [jax.ShapeDtypeStruct((n, _CH), jnp.float32),
                     jax.ShapeDtypeStruct((n, _CH), jnp.float32)]
    return pl.pallas_call(
        body, grid=grid, in_specs=in_specs, out_specs=out_specs,
        out_shape=out_shape)


def kernel(x, edge_index, lin_w, lin_b, conv_ws):
    n, d = x.shape
    e = edge_index.shape[1]
    h = lin_w.shape[0]
    blk = 2000

    npt = n // 16
    rows_t = e // (16 * _KS)
    src3d = edge_index[0].reshape(16, rows_t // _GRP, _GRP, _KS)
    dst3d = edge_index[1].reshape(16, rows_t // _GRP, _GRP, _KS)
    zeros_c = jnp.zeros((n, _CH), jnp.float32)
    dst_deg = edge_index[1].reshape(32, e // (32 * _KS), _KS)
    ones_c = jnp.ones((n, _CH), jnp.float32)

    spmm = _make_spmm_kernel(n, e)
    dega, degb = _make_deg_kernel(n, e)(dst_deg, zeros_c, ones_c)
    x0, lo, hi, dinv = _make_prologue(n, d, h, blk)(
        x, lin_w.T, lin_b.reshape(1, h), dega, degb)
    out = None
    for layer in range(_NUM_LAYERS):
        beta = math.log(_THETA / (layer + 1) + 1.0)
        alo, ahi = spmm(src3d, dst3d, lo, hi, zeros_c)
        last = layer == _NUM_LAYERS - 1
        layer_fn = _make_layer(n, h, beta, last, blk)
        args = (alo, ahi, lo, hi, x0, dinv, conv_ws[layer])
        if last:
            (out,) = layer_fn(*args)
        else:
            lo, hi = layer_fn(*args)
    return out


# speedup vs baseline: 1.5259x; 1.0016x over previous
import functools
import math

import jax
import jax.numpy as jnp
from jax import lax
from jax.experimental import pallas as pl
from jax.experimental.pallas import tpu as pltpu
from jax.experimental.pallas import tpu_sc as plsc

_ALPHA = 0.2
_THETA = 1.0
_NUM_LAYERS = 8

_KD = 40
_KS = 125
_WD = 16
_CH = 128


@functools.cache
def _make_deg_kernel(n_nodes: int, n_edges: int):
    rows_w = n_edges // (32 * _KS)
    npt = n_nodes // 16
    mesh = plsc.VectorSubcoreMesh(core_axis_name="c", subcore_axis_name="s")

    @functools.partial(
        pl.kernel,
        out_type=[jax.ShapeDtypeStruct((n_nodes, _CH), jnp.float32),
                  jax.ShapeDtypeStruct((n_nodes, _CH), jnp.float32)],
        mesh=mesh,
        scratch_types=[
            pltpu.VMEM((rows_w, _KS), jnp.int32),
            pltpu.VMEM((_KS, _CH), jnp.float32),
            pltpu.VMEM_SHARED((n_nodes, _CH), jnp.float32),
        ],
    )
    def deg_kernel(dst_hbm, z_hbm, ones_hbm, dega_hbm, degb_hbm,
                   idxv, ones_v, hist_sh):
        c = lax.axis_index("c")
        s = lax.axis_index("s")
        ali = -(-npt // 8) * 8
        tail = n_nodes - 15 * ali

        @pl.when(s < 15)
        def _():
            pltpu.sync_copy(z_hbm.at[pl.ds(s * ali, ali)],
                            hist_sh.at[pl.ds(s * ali, ali)])

        @pl.when(s == 15)
        def _():
            pltpu.sync_copy(z_hbm.at[pl.ds(15 * ali, tail)],
                            hist_sh.at[pl.ds(15 * ali, tail)])

        plsc.subcore_barrier()
        pltpu.sync_copy(dst_hbm.at[c * 16 + s], idxv)
        pltpu.sync_copy(ones_hbm.at[idxv.at[0]], ones_v)

        def chunk(i, carry):
            pltpu.sync_copy(ones_v, hist_sh.at[idxv.at[i]], add=True)
            return carry

        lax.fori_loop(0, rows_w, chunk, 0)
        plsc.subcore_barrier()

        def writeout(out_hbm):
            @pl.when(s < 15)
            def _():
                pltpu.sync_copy(hist_sh.at[pl.ds(s * ali, ali)],
                                out_hbm.at[pl.ds(s * ali, ali)])

            @pl.when(s == 15)
            def _():
                pltpu.sync_copy(hist_sh.at[pl.ds(15 * ali, tail)],
                                out_hbm.at[pl.ds(15 * ali, tail)])

        @pl.when(c == 0)
        def _():
            writeout(dega_hbm)

        @pl.when(c == 1)
        def _():
            writeout(degb_hbm)

    return deg_kernel


_GRP = 20


@functools.cache
def _make_spmm_kernel(n_nodes: int, n_edges: int):
    nrows = n_edges // _KS
    rows_t = nrows // 16
    ngrp = rows_t // _GRP
    npt = n_nodes // 16
    mesh = plsc.VectorSubcoreMesh(core_axis_name="c", subcore_axis_name="s")

    @functools.partial(
        pl.kernel,
        out_type=[jax.ShapeDtypeStruct((n_nodes, _CH), jnp.float32),
                  jax.ShapeDtypeStruct((n_nodes, _CH), jnp.float32)],
        mesh=mesh,
        scratch_types=[
            pltpu.VMEM((2, _GRP, _KS), jnp.int32),
            pltpu.VMEM((2, _GRP, _KS), jnp.int32),
            pltpu.VMEM((2, _KS, _CH), jnp.float32),
            pltpu.VMEM_SHARED((n_nodes, _CH), jnp.float32),
            pltpu.SemaphoreType.DMA((2,)),
            pltpu.SemaphoreType.DMA((2,)),
            pltpu.SemaphoreType.DMA((2,)),
        ],
    )
    def spmm_kernel(src_hbm, dst_hbm, lo_hbm, hi_hbm, z_hbm,
                    alo_hbm, ahi_hbm, srcv, dstv, rows_v, acc_sh,
                    isems, rsems, ssems):
        c = lax.axis_index("c")
        s = lax.axis_index("s")
        pltpu.async_copy(src_hbm.at[s].at[0], srcv.at[0], isems.at[0])
        pltpu.async_copy(dst_hbm.at[s].at[0], dstv.at[0], isems.at[0])
        ali = -(-npt // 8) * 8
        tail = n_nodes - 15 * ali

        @pl.when(s < 15)
        def _():
            pltpu.sync_copy(z_hbm.at[pl.ds(s * ali, ali)],
                            acc_sh.at[pl.ds(s * ali, ali)])

        @pl.when(s == 15)
        def _():
            pltpu.sync_copy(z_hbm.at[pl.ds(15 * ali, tail)],
                            acc_sh.at[pl.ds(15 * ali, tail)])

        plsc.subcore_barrier()

        def run(tab_hbm):
            def group(g, carry):
                gb = lax.rem(g, 2)
                gnb = lax.rem(g + 1, 2)

                @pl.when(g + 1 < ngrp)
                def _():
                    pltpu.async_copy(src_hbm.at[s].at[g + 1], srcv.at[gnb],
                                     isems.at[gnb])
                    pltpu.async_copy(dst_hbm.at[s].at[g + 1], dstv.at[gnb],
                                     isems.at[gnb])

                pltpu.make_async_copy(src_hbm.at[s].at[g], srcv.at[gb],
                                      isems.at[gb]).wait()
                pltpu.make_async_copy(dst_hbm.at[s].at[g], dstv.at[gb],
                                      isems.at[gb]).wait()
                pltpu.async_copy(tab_hbm.at[srcv.at[gb].at[0]], rows_v.at[0],
                                 rsems.at[0])

                def chunk(i, carry2):
                    b = lax.rem(i, 2)
                    nb = lax.rem(i + 1, 2)

                    @pl.when(i + 1 < _GRP)
                    def _():
                        pltpu.async_copy(tab_hbm.at[srcv.at[gb].at[i + 1]],
                                         rows_v.at[nb], rsems.at[nb])

                    pltpu.make_async_copy(tab_hbm.at[srcv.at[gb].at[i]],
                                          rows_v.at[b], rsems.at[b]).wait()
                    pltpu.sync_copy(rows_v.at[b],
                                    acc_sh.at[dstv.at[gb].at[i]], add=True)
                    return carry2

                lax.fori_loop(0, _GRP, chunk, 0)
                return carry

            lax.fori_loop(0, ngrp, group, 0)

        @pl.when(c == 0)
        def _():
            run(lo_hbm)

        @pl.when(c == 1)
        def _():
            run(hi_hbm)

        plsc.subcore_barrier()

        def writeout(dst_hbm):
            @pl.when(s < 15)
            def _():
                pltpu.sync_copy(acc_sh.at[pl.ds(s * ali, ali)],
                                dst_hbm.at[pl.ds(s * ali, ali)])

            @pl.when(s == 15)
            def _():
                pltpu.sync_copy(acc_sh.at[pl.ds(15 * ali, tail)],
                                dst_hbm.at[pl.ds(15 * ali, tail)])

        @pl.when(c == 0)
        def _():
            writeout(alo_hbm)

        @pl.when(c == 1)
        def _():
            writeout(ahi_hbm)

    return spmm_kernel


@functools.cache
def _make_inlin(n: int, d: int, h: int, blk: int):
    def body(x_ref, wt_ref, b_ref, x0_ref):
        hm = jnp.dot(x_ref[...], wt_ref[...],
                     preferred_element_type=jnp.float32) + b_ref[...]
        x0_ref[...] = jnp.maximum(hm, 0.0)

    return pl.pallas_call(
        body,
        grid=(n // blk,),
        in_specs=[
            pl.BlockSpec((blk, d), lambda i: (i, 0)),
            pl.BlockSpec((d, h), lambda i: (0, 0)),
            pl.BlockSpec((1, h), lambda i: (0, 0)),
        ],
        out_specs=[pl.BlockSpec((blk, h), lambda i: (i, 0))],
        out_shape=[jax.ShapeDtypeStruct((n, h), jnp.float32)],
    )


@functools.cache
def _make_scale(n: int, h: int, blk: int):
    def body(x0_ref, dega_ref, degb_ref, lo_ref, hi_ref, dinv_ref):
        deg = dega_ref[...] + degb_ref[...] + 1.0
        dinv = lax.rsqrt(deg)
        op = x0_ref[...] * dinv
        lo_ref[...] = op[:, :_CH]
        hi_ref[...] = op[:, _CH:]
        dinv_ref[...] = dinv

    return pl.pallas_call(
        body,
        grid=(n // blk,),
        in_specs=[
            pl.BlockSpec((blk, h), lambda i: (i, 0)),
            pl.BlockSpec((blk, 1), lambda i: (i, 0)),
            pl.BlockSpec((blk, 1), lambda i: (i, 0)),
        ],
        out_specs=[
            pl.BlockSpec((blk, _CH), lambda i: (i, 0)),
            pl.BlockSpec((blk, _CH), lambda i: (i, 0)),
            pl.BlockSpec((blk, 1), lambda i: (i, 0)),
        ],
        out_shape=[
            jax.ShapeDtypeStruct((n, _CH), jnp.float32),
            jax.ShapeDtypeStruct((n, _CH), jnp.float32),
            jax.ShapeDtypeStruct((n, 1), jnp.float32),
        ],
    )


@functools.cache
def _make_layer(n: int, h: int, beta: float, last: bool, blk: int):
    def body(alo_ref, ahi_ref, lo_ref, hi_ref, x0_ref, dinv_ref, w_ref, *outs):
        agg = jnp.concatenate(
            [alo_ref[...] + lo_ref[...], ahi_ref[...] + hi_ref[...]], axis=1)
        dinv = dinv_ref[...]
        hc = (1.0 - _ALPHA) * (agg * dinv) + _ALPHA * x0_ref[...]
        out = (1.0 - beta) * hc + beta * jnp.dot(
            hc, w_ref[...], preferred_element_type=jnp.float32)
        if last:
            outs[0][...] = out
        else:
            op = jnp.maximum(out, 0.0) * dinv
            outs[0][...] = op[:, :_CH]
            outs[1][...] = op[:, _CH:]

    grid = (n // blk,)
    in_specs = [
        pl.BlockSpec((blk, _CH), lambda i: (i, 0)),
        pl.BlockSpec((blk, _CH), lambda i: (i, 0)),
        pl.BlockSpec((blk, _CH), lambda i: (i, 0)),
        pl.BlockSpec((blk, _CH), lambda i: (i, 0)),
        pl.BlockSpec((blk, h), lambda i: (i, 0)),
        pl.BlockSpec((blk, 1), lambda i: (i, 0)),
        pl.BlockSpec((h, h), lambda i: (0, 0)),
    ]
    if last:
        out_specs = [pl.BlockSpec((blk, h), lambda i: (i, 0))]
        out_shape = [jax.ShapeDtypeStruct((n, h), jnp.float32)]
    else:
        out_specs = [pl.BlockSpec((blk, _CH), lambda i: (i, 0)),
                     pl.BlockSpec((blk, _CH), lambda i: (i, 0))]
        out_shape = [jax.ShapeDtypeStruct((n, _CH), jnp.float32),
                     jax.ShapeDtypeStruct((n, _CH), jnp.float32)]
    return pl.pallas_call(
        body, grid=grid, in_specs=in_specs, out_specs=out_specs,
        out_shape=out_shape)


def kernel(x, edge_index, lin_w, lin_b, conv_ws):
    n, d = x.shape
    e = edge_index.shape[1]
    h = lin_w.shape[0]
    blk = 2000

    npt = n // 16
    rows_t = e // (16 * _KS)
    src3d = edge_index[0].reshape(16, rows_t // _GRP, _GRP, _KS)
    dst3d = edge_index[1].reshape(16, rows_t // _GRP, _GRP, _KS)
    zeros_c = jnp.zeros((n, _CH), jnp.float32)
    dst_deg = edge_index[1].reshape(32, e // (32 * _KS), _KS)
    ones_c = jnp.ones((n, _CH), jnp.float32)

    spmm = _make_spmm_kernel(n, e)
    dega, degb = _make_deg_kernel(n, e)(dst_deg, zeros_c, ones_c)
    (x0,) = _make_inlin(n, d, h, blk)(x, lin_w.T, lin_b.reshape(1, h))
    lo, hi, dinv = _make_scale(n, h, blk)(
        x0, lax.slice(dega, (0, 0), (n, 1)), lax.slice(degb, (0, 0), (n, 1)))
    out = None
    for layer in range(_NUM_LAYERS):
        beta = math.log(_THETA / (layer + 1) + 1.0)
        alo, ahi = spmm(src3d, dst3d, lo, hi, zeros_c)
        last = layer == _NUM_LAYERS - 1
        layer_fn = _make_layer(n, h, beta, last, blk)
        args = (alo, ahi, lo, hi, x0, dinv, conv_ws[layer])
        if last:
            (out,) = layer_fn(*args)
        else:
            lo, hi = layer_fn(*args)
    return out
